# Initial kernel scaffold; baseline (speedup 1.0000x reference)
import jax
import jax.numpy as jnp
from jax.experimental import pallas as pl


def kernel(input_ids, token_table, pos_table):
    raise NotImplementedError("write your pallas kernel here")


# speedup vs baseline: 1.3161x; 1.3161x over previous
import functools

import jax
import jax.numpy as jnp
from jax import lax
from jax.experimental import pallas as pl
from jax.experimental.pallas import tpu as pltpu
from jax.experimental.pallas import tpu_sc as plsc

VOCAB = 100000
HIDDEN = 128
MAX_POS = 2048
B, T = 4, 2048
N_ROWS = B * T

_LANES = 16
_CHUNK = 128


def _make_sc_kernel():
    info = plsc.get_sparse_core_info()
    nc, ns = info.num_cores, info.num_subcores
    nw = nc * ns
    rows_w = N_ROWS // nw
    n_chunks = rows_w // _CHUNK

    mesh = plsc.VectorSubcoreMesh(core_axis_name="c", subcore_axis_name="s")

    @functools.partial(
        pl.kernel,
        mesh=mesh,
        out_type=jax.ShapeDtypeStruct((N_ROWS, HIDDEN), jnp.float32),
        scratch_types=[
            pltpu.VMEM((n_chunks, _CHUNK), jnp.int32),
            pltpu.VMEM((rows_w, HIDDEN), jnp.float32),
            pltpu.VMEM((rows_w, HIDDEN), jnp.float32),
            pltpu.SemaphoreType.DMA,
        ],
    )
    def sc_kernel(ids_hbm, tok_hbm, pos_hbm, out_hbm, idx_v, tok_v, pos_v, sem):
        wid = lax.axis_index("s") * nc + lax.axis_index("c")
        base = wid * rows_w
        pos_base = lax.rem(base, T)

        pltpu.sync_copy(ids_hbm.at[pl.ds(wid * n_chunks, n_chunks)], idx_v)

        copies = [
            pltpu.async_copy(
                tok_hbm.at[idx_v.at[c]],
                tok_v.at[pl.ds(c * _CHUNK, _CHUNK)],
                sem,
            )
            for c in range(n_chunks)
        ]
        pltpu.sync_copy(pos_hbm.at[pl.ds(pos_base, rows_w)], pos_v)
        for cp in copies:
            cp.wait()

        def body(i, carry):
            for j in range(HIDDEN // _LANES):
                sl = pl.ds(j * _LANES, _LANES)
                tok_v[i, sl] = tok_v[i, sl] + pos_v[i, sl]
            return carry

        lax.fori_loop(0, rows_w, body, 0, unroll=2)

        pltpu.sync_copy(tok_v, out_hbm.at[pl.ds(base, rows_w)])

    return sc_kernel


def kernel(input_ids, token_table, pos_table):
    ids_flat = input_ids.reshape(N_ROWS // _CHUNK, _CHUNK).astype(jnp.int32)
    out = _make_sc_kernel()(ids_flat, token_table, pos_table)
    return out.reshape(B, T, HIDDEN)


# speedup vs baseline: 1.7549x; 1.3333x over previous
import functools

import jax
import jax.numpy as jnp
from jax import lax
from jax.experimental import pallas as pl
from jax.experimental.pallas import tpu as pltpu
from jax.experimental.pallas import tpu_sc as plsc

VOCAB = 100000
HIDDEN = 128
MAX_POS = 2048
B, T = 4, 2048
N_ROWS = B * T

_LANES = 16
_CHUNK = 128


def _make_sc_kernel():
    info = plsc.get_sparse_core_info()
    nc, ns = info.num_cores, info.num_subcores
    nw = nc * ns
    rows_w = N_ROWS // nw
    n_chunks = rows_w // _CHUNK

    mesh = plsc.VectorSubcoreMesh(core_axis_name="c", subcore_axis_name="s")

    @functools.partial(
        pl.kernel,
        mesh=mesh,
        out_type=jax.ShapeDtypeStruct((N_ROWS, HIDDEN), jnp.float32),
        scratch_types=[
            pltpu.VMEM((n_chunks, _CHUNK), jnp.int32),
            pltpu.VMEM((rows_w, HIDDEN), jnp.float32),
            pltpu.SemaphoreType.DMA,
        ],
    )
    def sc_kernel(ids_hbm, tok_hbm, pos_hbm, out_hbm, idx_v, tok_v, sem):
        wid = lax.axis_index("s") * nc + lax.axis_index("c")
        base = wid * rows_w
        pos_base = lax.rem(base, T)

        pltpu.sync_copy(ids_hbm.at[pl.ds(wid * n_chunks, n_chunks)], idx_v)
        pltpu.sync_copy(pos_hbm.at[pl.ds(pos_base, rows_w)], tok_v)

        copies = [
            pltpu.async_copy(
                tok_hbm.at[idx_v.at[c]],
                tok_v.at[pl.ds(c * _CHUNK, _CHUNK)],
                sem,
                add=True,
            )
            for c in range(n_chunks)
        ]
        for cp in copies:
            cp.wait()

        pltpu.sync_copy(tok_v, out_hbm.at[pl.ds(base, rows_w)])

    return sc_kernel


def kernel(input_ids, token_table, pos_table):
    ids_flat = input_ids.reshape(N_ROWS // _CHUNK, _CHUNK).astype(jnp.int32)
    out = _make_sc_kernel()(ids_flat, token_table, pos_table)
    return out.reshape(B, T, HIDDEN)


# speedup vs baseline: 1.7649x; 1.0057x over previous
import functools

import jax
import jax.numpy as jnp
from jax import lax
from jax.experimental import pallas as pl
from jax.experimental.pallas import tpu as pltpu
from jax.experimental.pallas import tpu_sc as plsc

VOCAB = 100000
HIDDEN = 128
MAX_POS = 2048
B, T = 4, 2048
N_ROWS = B * T

_N_CHUNKS = 4
_CHUNK = 64


def _make_sc_kernel():
    info = plsc.get_sparse_core_info()
    nc, ns = info.num_cores, info.num_subcores
    nw = nc * ns
    rows_w = N_ROWS // nw
    assert rows_w == _N_CHUNKS * _CHUNK

    mesh = plsc.VectorSubcoreMesh(core_axis_name="c", subcore_axis_name="s")

    @functools.partial(
        pl.kernel,
        mesh=mesh,
        out_type=jax.ShapeDtypeStruct((N_ROWS, HIDDEN), jnp.float32),
        scratch_types=[
            pltpu.VMEM((_N_CHUNKS, _CHUNK), jnp.int32),
            pltpu.VMEM((rows_w, HIDDEN), jnp.float32),
        ]
        + [pltpu.SemaphoreType.DMA] * (3 * _N_CHUNKS),
    )
    def sc_kernel(ids_hbm, tok_hbm, pos_hbm, out_hbm, idx_v, tok_v, *sems):
        sem_p = sems[:_N_CHUNKS]
        sem_g = sems[_N_CHUNKS : 2 * _N_CHUNKS]
        sem_o = sems[2 * _N_CHUNKS :]

        wid = lax.axis_index("s") * nc + lax.axis_index("c")
        base = wid * rows_w
        pos_base = lax.rem(base, T)

        pltpu.sync_copy(ids_hbm.at[pl.ds(wid * _N_CHUNKS, _N_CHUNKS)], idx_v)

        def chunk(ref, c):
            return ref.at[pl.ds(c * _CHUNK, _CHUNK)]

        pos_cp = [
            pltpu.async_copy(
                pos_hbm.at[pl.ds(pos_base + c * _CHUNK, _CHUNK)],
                chunk(tok_v, c),
                sem_p[c],
            )
            for c in range(_N_CHUNKS)
        ]
        g_cp = []
        for c in range(_N_CHUNKS):
            pos_cp[c].wait()
            g_cp.append(
                pltpu.async_copy(
                    tok_hbm.at[idx_v.at[c]], chunk(tok_v, c), sem_g[c], add=True
                )
            )
        out_cp = []
        for c in range(_N_CHUNKS):
            g_cp[c].wait()
            out_cp.append(
                pltpu.async_copy(
                    chunk(tok_v, c),
                    out_hbm.at[pl.ds(base + c * _CHUNK, _CHUNK)],
                    sem_o[c],
                )
            )
        for cp in out_cp:
            cp.wait()

    return sc_kernel


def kernel(input_ids, token_table, pos_table):
    ids_flat = input_ids.reshape(N_ROWS // _CHUNK, _CHUNK).astype(jnp.int32)
    out = _make_sc_kernel()(ids_flat, token_table, pos_table)
    return out.reshape(B, T, HIDDEN)
